# baseline (device time: 125945 ns/iter reference)
import jax
import jax.numpy as jnp
from jax import lax
from jax.experimental import pallas as pl
from jax.experimental.pallas import tpu as pltpu

N_DEV = 16
B, SQ, D = 2, 128, 512
HQ_LOC, DH = 8, 64
HKV_LOC = 2
R = B * SQ


def kernel(x, Wq, Wo, Wk, Wv):
    my_pos = lax.axis_index("i")
    kv_col = my_pos * (HKV_LOC * DH)
    Wk_sl = lax.dynamic_slice(Wk, (0, kv_col), (D, HKV_LOC * DH))
    Wv_sl = lax.dynamic_slice(Wv, (0, kv_col), (D, HKV_LOC * DH))

    def body(x_ref, wq_ref, wo_ref, wk_ref, wv_ref, out_ref,
             comm_ref, send_sems, recv_sems):
        i = lax.axis_index("i")
        left = lax.rem(i - 1 + N_DEV, N_DEV)
        right = lax.rem(i + 1, N_DEV)

        barrier = pltpu.get_barrier_semaphore()
        for nbr in (left, right):
            pl.semaphore_signal(
                barrier, inc=1,
                device_id=(nbr,), device_id_type=pl.DeviceIdType.MESH,
            )
        pl.semaphore_wait(barrier, 2)

        xf = x_ref[...].reshape(R, D)
        q = jnp.dot(xf, wq_ref[...], preferred_element_type=jnp.float32)
        k = jnp.dot(xf, wk_ref[...], preferred_element_type=jnp.float32)
        v = jnp.dot(xf, wv_ref[...], preferred_element_type=jnp.float32)

        outs = []
        for b in range(B):
            rows = slice(b * SQ, (b + 1) * SQ)
            head_outs = []
            for h in range(HQ_LOC):
                g = h // 4
                qbh = q[rows, h * DH:(h + 1) * DH]
                kbg = k[rows, g * DH:(g + 1) * DH]
                vbg = v[rows, g * DH:(g + 1) * DH]
                s = lax.dot_general(
                    qbh, kbg, (((1,), (1,)), ((), ())),
                    preferred_element_type=jnp.float32,
                ) * 0.125
                m = jnp.max(s, axis=-1, keepdims=True)
                p = jnp.exp(s - m)
                l = jnp.sum(p, axis=-1, keepdims=True)
                o = jnp.dot(p, vbg, preferred_element_type=jnp.float32) / l
                head_outs.append(o)
            outs.append(jnp.concatenate(head_outs, axis=1))
        att = jnp.concatenate(outs, axis=0)

        partial = jnp.dot(att, wo_ref[...], preferred_element_type=jnp.float32)

        acc = partial
        comm_ref[0] = partial
        for hop in range(N_DEV - 1):
            send_slot = hop % 2
            recv_slot = (hop + 1) % 2
            rdma = pltpu.make_async_remote_copy(
                src_ref=comm_ref.at[send_slot],
                dst_ref=comm_ref.at[recv_slot],
                send_sem=send_sems.at[send_slot],
                recv_sem=recv_sems.at[recv_slot],
                device_id=(right,),
                device_id_type=pl.DeviceIdType.MESH,
            )
            rdma.start()
            rdma.wait()
            acc = acc + comm_ref[recv_slot]
        out_ref[...] = acc.reshape(B, SQ, D)

    return pl.pallas_call(
        body,
        out_shape=jax.ShapeDtypeStruct((B, SQ, D), jnp.float32),
        in_specs=[pl.BlockSpec(memory_space=pltpu.VMEM)] * 5,
        out_specs=pl.BlockSpec(memory_space=pltpu.VMEM),
        scratch_shapes=[
            pltpu.VMEM((2, R, D), jnp.float32),
            pltpu.SemaphoreType.DMA((2,)),
            pltpu.SemaphoreType.DMA((2,)),
        ],
        compiler_params=pltpu.CompilerParams(collective_id=0),
    )(x, Wq, Wo, Wk_sl, Wv_sl)


# device time: 44929 ns/iter; 2.8032x vs baseline; 2.8032x over previous
import jax
import jax.numpy as jnp
from jax import lax
from jax.experimental import pallas as pl
from jax.experimental.pallas import tpu as pltpu

N_DEV = 16
B, SQ, D = 2, 128, 512
HQ_LOC, DH = 8, 64
HKV_LOC = 2
R = B * SQ


def kernel(x, Wq, Wo, Wk, Wv):
    my_pos = lax.axis_index("i")
    kv_col = my_pos * (HKV_LOC * DH)
    Wk_sl = lax.dynamic_slice(Wk, (0, kv_col), (D, HKV_LOC * DH))
    Wv_sl = lax.dynamic_slice(Wv, (0, kv_col), (D, HKV_LOC * DH))

    def body(x_ref, wq_ref, wo_ref, wk_ref, wv_ref, out_ref,
             acc_ref, recv_ref, send_sems, recv_sems):
        i = lax.axis_index("i")

        barrier = pltpu.get_barrier_semaphore()
        for d in (1, 2, 4, 8):
            pl.semaphore_signal(
                barrier, inc=1,
                device_id=(jnp.bitwise_xor(i, d),),
                device_id_type=pl.DeviceIdType.MESH,
            )
        pl.semaphore_wait(barrier, 4)

        xf = x_ref[...].reshape(R, D)
        q = jnp.dot(xf, wq_ref[...], preferred_element_type=jnp.float32)
        k = jnp.dot(xf, wk_ref[...], preferred_element_type=jnp.float32)
        v = jnp.dot(xf, wv_ref[...], preferred_element_type=jnp.float32)

        outs = []
        for b in range(B):
            rows = slice(b * SQ, (b + 1) * SQ)
            head_outs = []
            for h in range(HQ_LOC):
                g = h // 4
                qbh = q[rows, h * DH:(h + 1) * DH]
                kbg = k[rows, g * DH:(g + 1) * DH]
                vbg = v[rows, g * DH:(g + 1) * DH]
                s = lax.dot_general(
                    qbh, kbg, (((1,), (1,)), ((), ())),
                    preferred_element_type=jnp.float32,
                ) * 0.125
                m = jnp.max(s, axis=-1, keepdims=True)
                p = jnp.exp(s - m)
                l = jnp.sum(p, axis=-1, keepdims=True)
                o = jnp.dot(p, vbg, preferred_element_type=jnp.float32) / l
                head_outs.append(o)
            outs.append(jnp.concatenate(head_outs, axis=1))
        att = jnp.concatenate(outs, axis=0)

        partial = jnp.dot(att, wo_ref[...], preferred_element_type=jnp.float32)

        acc_ref[...] = partial

        RS_OFF = {3: 0, 2: 128, 1: 192, 0: 224}
        s = jnp.int32(0)
        L = R
        step = 0
        for b in (3, 2, 1, 0):
            half = L // 2
            partner = jnp.bitwise_xor(i, 1 << b)
            bit = (i >> b) & 1
            send_start = s + jnp.where(bit == 0, half, 0)
            keep_start = s + jnp.where(bit == 0, 0, half)
            rdma = pltpu.make_async_remote_copy(
                src_ref=acc_ref.at[pl.ds(send_start, half)],
                dst_ref=recv_ref.at[pl.ds(RS_OFF[b], half)],
                send_sem=send_sems.at[step],
                recv_sem=recv_sems.at[step],
                device_id=(partner,),
                device_id_type=pl.DeviceIdType.MESH,
            )
            rdma.start()
            rdma.wait()
            acc_ref[pl.ds(keep_start, half)] = (
                acc_ref[pl.ds(keep_start, half)]
                + recv_ref[pl.ds(RS_OFF[b], half)]
            )
            s = keep_start
            L = half
            step += 1

        for b in (0, 1, 2, 3):
            partner = jnp.bitwise_xor(i, 1 << b)
            bit = (i >> b) & 1
            rdma = pltpu.make_async_remote_copy(
                src_ref=acc_ref.at[pl.ds(s, L)],
                dst_ref=acc_ref.at[pl.ds(s, L)],
                send_sem=send_sems.at[step],
                recv_sem=recv_sems.at[step],
                device_id=(partner,),
                device_id_type=pl.DeviceIdType.MESH,
            )
            rdma.start()
            rdma.wait()
            s = s - bit * L
            L = L * 2
            step += 1

        out_ref[...] = acc_ref[...].reshape(B, SQ, D)

    return pl.pallas_call(
        body,
        out_shape=jax.ShapeDtypeStruct((B, SQ, D), jnp.float32),
        in_specs=[pl.BlockSpec(memory_space=pltpu.VMEM)] * 5,
        out_specs=pl.BlockSpec(memory_space=pltpu.VMEM),
        scratch_shapes=[
            pltpu.VMEM((R, D), jnp.float32),
            pltpu.VMEM((R, D), jnp.float32),
            pltpu.SemaphoreType.DMA((8,)),
            pltpu.SemaphoreType.DMA((8,)),
        ],
        compiler_params=pltpu.CompilerParams(collective_id=0),
    )(x, Wq, Wo, Wk_sl, Wv_sl)


# device time: 27659 ns/iter; 4.5535x vs baseline; 1.6244x over previous
import jax
import jax.numpy as jnp
from jax import lax
from jax.experimental import pallas as pl
from jax.experimental.pallas import tpu as pltpu

N_DEV = 16
B, SQ, D = 2, 128, 512
HQ_LOC, DH = 8, 64
HKV_LOC = 2
R = B * SQ
CH = R // N_DEV


def kernel(x, Wq, Wo, Wk, Wv):
    my_pos = lax.axis_index("i")
    kv_col = my_pos * (HKV_LOC * DH)
    Wk_sl = lax.dynamic_slice(Wk, (0, kv_col), (D, HKV_LOC * DH))
    Wv_sl = lax.dynamic_slice(Wv, (0, kv_col), (D, HKV_LOC * DH))

    def body(x_ref, wq_ref, wo_ref, wk_ref, wv_ref, out_ref,
             acc_ref, stage_ref, s1_send, s1_recv, s2_send, s2_recv):
        i = lax.axis_index("i")

        barrier = pltpu.get_barrier_semaphore()
        for r in range(1, N_DEV):
            pl.semaphore_signal(
                barrier, inc=1,
                device_id=(lax.rem(i + r, N_DEV),),
                device_id_type=pl.DeviceIdType.MESH,
            )
        pl.semaphore_wait(barrier, N_DEV - 1)

        xf = x_ref[...].reshape(R, D)
        q = jnp.dot(xf, wq_ref[...], preferred_element_type=jnp.float32)
        k = jnp.dot(xf, wk_ref[...], preferred_element_type=jnp.float32)
        v = jnp.dot(xf, wv_ref[...], preferred_element_type=jnp.float32)

        outs = []
        for b in range(B):
            rows = slice(b * SQ, (b + 1) * SQ)
            head_outs = []
            for g in range(HKV_LOC):
                qcat = jnp.concatenate(
                    [q[rows, h * DH:(h + 1) * DH] for h in range(4 * g, 4 * g + 4)],
                    axis=0,
                )
                kbg = k[rows, g * DH:(g + 1) * DH]
                vbg = v[rows, g * DH:(g + 1) * DH]
                s = lax.dot_general(
                    qcat, kbg, (((1,), (1,)), ((), ())),
                    preferred_element_type=jnp.float32,
                ) * 0.125
                m = jnp.max(s, axis=-1, keepdims=True)
                p = jnp.exp(s - m)
                l = jnp.sum(p, axis=-1, keepdims=True)
                o = jnp.dot(p, vbg, preferred_element_type=jnp.float32) / l
                head_outs.extend(o[h * SQ:(h + 1) * SQ, :] for h in range(4))
            outs.append(jnp.concatenate(head_outs, axis=1))
        att = jnp.concatenate(outs, axis=0)

        acc_ref[...] = jnp.dot(att, wo_ref[...],
                               preferred_element_type=jnp.float32)

        descs1 = []
        for r in range(1, N_DEV):
            dest = lax.rem(i + r, N_DEV)
            d = pltpu.make_async_remote_copy(
                src_ref=acc_ref.at[pl.ds(dest * CH, CH)],
                dst_ref=stage_ref.at[r - 1],
                send_sem=s1_send.at[r - 1],
                recv_sem=s1_recv.at[r - 1],
                device_id=(dest,),
                device_id_type=pl.DeviceIdType.MESH,
            )
            d.start()
            descs1.append(d)

        red = acc_ref[pl.ds(i * CH, CH)]
        for r, d in enumerate(descs1, start=1):
            d.wait_recv()
            red = red + stage_ref[r - 1]
        for d in descs1:
            d.wait_send()
        acc_ref[pl.ds(i * CH, CH)] = red

        descs2 = []
        for r in range(1, N_DEV):
            dest = lax.rem(i + r, N_DEV)
            d = pltpu.make_async_remote_copy(
                src_ref=acc_ref.at[pl.ds(i * CH, CH)],
                dst_ref=acc_ref.at[pl.ds(i * CH, CH)],
                send_sem=s2_send.at[r - 1],
                recv_sem=s2_recv.at[r - 1],
                device_id=(dest,),
                device_id_type=pl.DeviceIdType.MESH,
            )
            d.start()
            descs2.append(d)
        for d in descs2:
            d.wait_recv()
        for d in descs2:
            d.wait_send()

        out_ref[...] = acc_ref[...].reshape(B, SQ, D)

    return pl.pallas_call(
        body,
        out_shape=jax.ShapeDtypeStruct((B, SQ, D), jnp.float32),
        in_specs=[pl.BlockSpec(memory_space=pltpu.VMEM)] * 5,
        out_specs=pl.BlockSpec(memory_space=pltpu.VMEM),
        scratch_shapes=[
            pltpu.VMEM((R, D), jnp.float32),
            pltpu.VMEM((N_DEV - 1, CH, D), jnp.float32),
            pltpu.SemaphoreType.DMA((N_DEV - 1,)),
            pltpu.SemaphoreType.DMA((N_DEV - 1,)),
            pltpu.SemaphoreType.DMA((N_DEV - 1,)),
            pltpu.SemaphoreType.DMA((N_DEV - 1,)),
        ],
        compiler_params=pltpu.CompilerParams(collective_id=0),
    )(x, Wq, Wo, Wk_sl, Wv_sl)


# device time: 24286 ns/iter; 5.1859x vs baseline; 1.1389x over previous
import jax
import jax.numpy as jnp
from jax import lax
from jax.experimental import pallas as pl
from jax.experimental.pallas import tpu as pltpu

N_DEV = 16
B, SQ, D = 2, 128, 512
HQ_LOC, DH = 8, 64
HKV_LOC = 2
R = B * SQ
CH = R // N_DEV


def kernel(x, Wq, Wo, Wk, Wv):
    my_pos = lax.axis_index("i")
    kv_col = my_pos * (HKV_LOC * DH)
    Wk_sl = lax.dynamic_slice(Wk, (0, kv_col), (D, HKV_LOC * DH))
    Wv_sl = lax.dynamic_slice(Wv, (0, kv_col), (D, HKV_LOC * DH))

    def body(x_ref, wq_ref, wo_ref, wk_ref, wv_ref, out_ref,
             acc_ref, stage_ref, s1_send, s1_recv, s2_send, s2_recv):
        i = lax.axis_index("i")

        barrier = pltpu.get_barrier_semaphore()
        for r in range(1, N_DEV):
            pl.semaphore_signal(
                barrier, inc=1,
                device_id=(lax.rem(i + r, N_DEV),),
                device_id_type=pl.DeviceIdType.MESH,
            )
        pl.semaphore_wait(barrier, N_DEV - 1)

        bf16 = jnp.bfloat16
        xf = x_ref[...].reshape(R, D).astype(bf16)
        q = jnp.dot(xf, wq_ref[...].astype(bf16),
                    preferred_element_type=jnp.float32)
        k = jnp.dot(xf, wk_ref[...].astype(bf16),
                    preferred_element_type=jnp.float32)
        v = jnp.dot(xf, wv_ref[...].astype(bf16),
                    preferred_element_type=jnp.float32)

        outs = []
        for b in range(B):
            rows = slice(b * SQ, (b + 1) * SQ)
            head_outs = []
            for g in range(HKV_LOC):
                qcat = jnp.concatenate(
                    [q[rows, h * DH:(h + 1) * DH] for h in range(4 * g, 4 * g + 4)],
                    axis=0,
                ).astype(bf16)
                kbg = k[rows, g * DH:(g + 1) * DH].astype(bf16)
                vbg = v[rows, g * DH:(g + 1) * DH].astype(bf16)
                s = lax.dot_general(
                    qcat, kbg, (((1,), (1,)), ((), ())),
                    preferred_element_type=jnp.float32,
                ) * 0.125
                m = jnp.max(s, axis=-1, keepdims=True)
                p = jnp.exp(s - m)
                l = jnp.sum(p, axis=-1, keepdims=True)
                o = jnp.dot(p.astype(bf16), vbg,
                            preferred_element_type=jnp.float32) / l
                head_outs.extend(o[h * SQ:(h + 1) * SQ, :] for h in range(4))
            outs.append(jnp.concatenate(head_outs, axis=1))
        att = jnp.concatenate(outs, axis=0)

        acc_ref[...] = jnp.dot(att.astype(bf16), wo_ref[...].astype(bf16),
                               preferred_element_type=jnp.float32).astype(bf16)

        descs1 = []
        for r in range(1, N_DEV):
            dest = lax.rem(i + r, N_DEV)
            d = pltpu.make_async_remote_copy(
                src_ref=acc_ref.at[pl.ds(dest * CH, CH)],
                dst_ref=stage_ref.at[r - 1],
                send_sem=s1_send.at[r - 1],
                recv_sem=s1_recv.at[r - 1],
                device_id=(dest,),
                device_id_type=pl.DeviceIdType.MESH,
            )
            d.start()
            descs1.append(d)

        red = acc_ref[pl.ds(i * CH, CH)].astype(jnp.float32)
        for r, d in enumerate(descs1, start=1):
            d.wait_recv()
            red = red + stage_ref[r - 1].astype(jnp.float32)
        for d in descs1:
            d.wait_send()
        acc_ref[pl.ds(i * CH, CH)] = red.astype(bf16)

        descs2 = []
        for r in range(1, N_DEV):
            dest = lax.rem(i + r, N_DEV)
            d = pltpu.make_async_remote_copy(
                src_ref=acc_ref.at[pl.ds(i * CH, CH)],
                dst_ref=acc_ref.at[pl.ds(i * CH, CH)],
                send_sem=s2_send.at[r - 1],
                recv_sem=s2_recv.at[r - 1],
                device_id=(dest,),
                device_id_type=pl.DeviceIdType.MESH,
            )
            d.start()
            descs2.append(d)
        for d in descs2:
            d.wait_recv()
        for d in descs2:
            d.wait_send()

        out_ref[...] = acc_ref[...].astype(jnp.float32).reshape(B, SQ, D)

    return pl.pallas_call(
        body,
        out_shape=jax.ShapeDtypeStruct((B, SQ, D), jnp.float32),
        in_specs=[pl.BlockSpec(memory_space=pltpu.VMEM)] * 5,
        out_specs=pl.BlockSpec(memory_space=pltpu.VMEM),
        scratch_shapes=[
            pltpu.VMEM((R, D), jnp.bfloat16),
            pltpu.VMEM((N_DEV - 1, CH, D), jnp.bfloat16),
            pltpu.SemaphoreType.DMA((N_DEV - 1,)),
            pltpu.SemaphoreType.DMA((N_DEV - 1,)),
            pltpu.SemaphoreType.DMA((N_DEV - 1,)),
            pltpu.SemaphoreType.DMA((N_DEV - 1,)),
        ],
        compiler_params=pltpu.CompilerParams(collective_id=0),
    )(x, Wq, Wo, Wk_sl, Wv_sl)


# device time: 24034 ns/iter; 5.2403x vs baseline; 1.0105x over previous
import jax
import jax.numpy as jnp
from jax import lax
from jax.experimental import pallas as pl
from jax.experimental.pallas import tpu as pltpu

N_DEV = 16
B, SQ, D = 2, 128, 512
HQ_LOC, DH = 8, 64
HKV_LOC = 2
R = B * SQ
CH = R // N_DEV


def kernel(x, Wq, Wo, Wk, Wv):
    my_pos = lax.axis_index("i")
    kv_col = my_pos * (HKV_LOC * DH)
    Wk_sl = lax.dynamic_slice(Wk, (0, kv_col), (D, HKV_LOC * DH))
    Wv_sl = lax.dynamic_slice(Wv, (0, kv_col), (D, HKV_LOC * DH))

    def body(x_ref, wq_ref, wo_ref, wk_ref, wv_ref, out_ref,
             acc_ref, stage_ref, s1_send, s1_recv, s2_send, s2_recv):
        i = lax.axis_index("i")

        barrier = pltpu.get_barrier_semaphore()
        for r in range(1, N_DEV):
            pl.semaphore_signal(
                barrier, inc=1,
                device_id=(lax.rem(i + r, N_DEV),),
                device_id_type=pl.DeviceIdType.MESH,
            )
        pl.semaphore_wait(barrier, N_DEV - 1)

        bf16 = jnp.bfloat16
        xf = x_ref[...].reshape(R, D).astype(bf16)
        q = jnp.dot(xf, wq_ref[...].astype(bf16),
                    preferred_element_type=jnp.float32)
        k = jnp.dot(xf, wk_ref[...].astype(bf16),
                    preferred_element_type=jnp.float32)
        v = jnp.dot(xf, wv_ref[...].astype(bf16),
                    preferred_element_type=jnp.float32)

        wo = wo_ref[...].astype(bf16)
        descs1 = []
        for r in range(1, N_DEV):
            dest = lax.rem(i + r, N_DEV)
            descs1.append((dest, pltpu.make_async_remote_copy(
                src_ref=acc_ref.at[pl.ds(dest * CH, CH)],
                dst_ref=stage_ref.at[r - 1],
                send_sem=s1_send.at[r - 1],
                recv_sem=s1_recv.at[r - 1],
                device_id=(dest,),
                device_id_type=pl.DeviceIdType.MESH,
            )))

        for b in range(B):
            rows = slice(b * SQ, (b + 1) * SQ)
            head_outs = []
            for g in range(HKV_LOC):
                qcat = jnp.concatenate(
                    [q[rows, h * DH:(h + 1) * DH] for h in range(4 * g, 4 * g + 4)],
                    axis=0,
                ).astype(bf16)
                kbg = k[rows, g * DH:(g + 1) * DH].astype(bf16)
                vbg = v[rows, g * DH:(g + 1) * DH].astype(bf16)
                s = lax.dot_general(
                    qcat, kbg, (((1,), (1,)), ((), ())),
                    preferred_element_type=jnp.float32,
                ) * 0.125
                m = jnp.max(s, axis=-1, keepdims=True)
                p = jnp.exp(s - m)
                l = jnp.sum(p, axis=-1, keepdims=True)
                o = jnp.dot(p.astype(bf16), vbg,
                            preferred_element_type=jnp.float32) / l
                head_outs.extend(o[h * SQ:(h + 1) * SQ, :] for h in range(4))
            att_b = jnp.concatenate(head_outs, axis=1).astype(bf16)
            acc_ref[pl.ds(b * SQ, SQ)] = jnp.dot(
                att_b, wo, preferred_element_type=jnp.float32
            ).astype(bf16)
            for dest, d in descs1:
                @pl.when(dest // (SQ // CH) == b)
                def _(d=d):
                    d.start()

        red = acc_ref[pl.ds(i * CH, CH)].astype(jnp.float32)
        for r, (dest, d) in enumerate(descs1, start=1):
            d.wait_recv()
        red = red + jnp.sum(stage_ref[...].astype(jnp.float32), axis=0)
        for dest, d in descs1:
            d.wait_send()
        acc_ref[pl.ds(i * CH, CH)] = red.astype(bf16)

        descs2 = []
        for r in range(1, N_DEV):
            dest = lax.rem(i + r, N_DEV)
            d = pltpu.make_async_remote_copy(
                src_ref=acc_ref.at[pl.ds(i * CH, CH)],
                dst_ref=acc_ref.at[pl.ds(i * CH, CH)],
                send_sem=s2_send.at[r - 1],
                recv_sem=s2_recv.at[r - 1],
                device_id=(dest,),
                device_id_type=pl.DeviceIdType.MESH,
            )
            d.start()
            descs2.append(d)
        for d in descs2:
            d.wait_recv()
        for d in descs2:
            d.wait_send()

        out_ref[...] = acc_ref[...].astype(jnp.float32).reshape(B, SQ, D)

    return pl.pallas_call(
        body,
        out_shape=jax.ShapeDtypeStruct((B, SQ, D), jnp.float32),
        in_specs=[pl.BlockSpec(memory_space=pltpu.VMEM)] * 5,
        out_specs=pl.BlockSpec(memory_space=pltpu.VMEM),
        scratch_shapes=[
            pltpu.VMEM((R, D), jnp.bfloat16),
            pltpu.VMEM((N_DEV - 1, CH, D), jnp.bfloat16),
            pltpu.SemaphoreType.DMA((N_DEV - 1,)),
            pltpu.SemaphoreType.DMA((N_DEV - 1,)),
            pltpu.SemaphoreType.DMA((N_DEV - 1,)),
            pltpu.SemaphoreType.DMA((N_DEV - 1,)),
        ],
        compiler_params=pltpu.CompilerParams(collective_id=0),
    )(x, Wq, Wo, Wk_sl, Wv_sl)
